# baseline (speedup 1.0000x reference)
import jax
import jax.numpy as jnp
from jax import lax
from jax.experimental import pallas as pl
from jax.experimental.pallas import tpu as pltpu
from jax.experimental.pallas import tpu_sc as plsc

NC, NS, LANES = 2, 16, 16
NW = NC * NS
SEQ = 128
D = 64
B = 4096
SEQ_PER_W = B // NW
SEQ_PER_CHUNK = 4
CHUNK = SEQ_PER_CHUNK * SEQ
NCHUNK = SEQ_PER_W // SEQ_PER_CHUNK
ROWS_PER_W = SEQ_PER_W * SEQ
DV = D // LANES


def _emb_kernel(x_hbm, table_hbm, pos_hbm, out_hbm, idx_v, pos_v, rows_v,
                sem0, sem1):
    cid = lax.axis_index("c")
    sid = lax.axis_index("s")
    wid = sid * NC + cid
    seq_base = pl.multiple_of(wid * SEQ_PER_W, SEQ_PER_W)
    row_base = pl.multiple_of(wid * ROWS_PER_W, ROWS_PER_W)

    pltpu.sync_copy(x_hbm.at[pl.ds(seq_base, SEQ_PER_W)], idx_v)
    pltpu.sync_copy(pos_hbm, pos_v)

    sems = (sem0, sem1)

    def fire(g, buf):
        for s in range(SEQ_PER_CHUNK):
            seq = g * SEQ_PER_CHUNK + s
            pltpu.async_copy(
                table_hbm.at[idx_v.at[seq]],
                rows_v.at[buf, pl.ds(s * SEQ, SEQ)],
                sems[buf],
            )

    def drain(buf):
        pltpu.make_async_copy(
            out_hbm.at[pl.ds(0, CHUNK)], rows_v.at[buf], sems[buf]
        ).wait()

    def add_pos(buf):
        def body(p, _):
            for c in range(DV):
                v = pos_v[p, pl.ds(c * LANES, LANES)]
                for s in range(SEQ_PER_CHUNK):
                    plsc.addupdate(
                        rows_v.at[buf, s * SEQ + p, pl.ds(c * LANES, LANES)],
                        v,
                    )
            return 0
        lax.fori_loop(0, SEQ, body, 0)

    def store(g, buf):
        start = pl.multiple_of(row_base + g * CHUNK, CHUNK)
        pltpu.sync_copy(rows_v.at[buf], out_hbm.at[pl.ds(start, CHUNK)])

    fire(0, 0)

    @pl.loop(0, NCHUNK, step=2)
    def _chunks(g0):
        for b in range(2):
            g = g0 + b

            @pl.when(g + 1 < NCHUNK)
            def _():
                fire(g + 1, 1 - b)

            drain(b)
            add_pos(b)
            store(g, b)


def kernel(x, table, pos):
    mesh = plsc.VectorSubcoreMesh(
        core_axis_name="c", subcore_axis_name="s",
        num_cores=NC, num_subcores=NS,
    )
    f = pl.kernel(
        _emb_kernel,
        out_type=jax.ShapeDtypeStruct((B * SEQ, D), jnp.float32),
        mesh=mesh,
        scratch_types=[
            pltpu.VMEM((SEQ_PER_W, SEQ), jnp.int32),
            pltpu.VMEM((SEQ, D), jnp.float32),
            pltpu.VMEM((2, CHUNK, D), jnp.float32),
            pltpu.SemaphoreType.DMA,
            pltpu.SemaphoreType.DMA,
        ],
        compiler_params=pltpu.CompilerParams(use_tc_tiling_on_sc=False),
    )
    out = f(x, table, pos)
    return out.reshape(B, SEQ, D)
